# Initial kernel scaffold; baseline (speedup 1.0000x reference)
import jax
import jax.numpy as jnp
from jax.experimental import pallas as pl


def kernel(x, edge_index, batch, W1, b1, W2, b2, W3, b3, Wl, bl):
    raise NotImplementedError("write your pallas kernel here")


# speedup vs baseline: 15.1251x; 15.1251x over previous
import functools

import jax
import jax.numpy as jnp
from jax import lax
from jax.experimental import pallas as pl
from jax.experimental.pallas import tpu as pltpu
from jax.experimental.pallas import tpu_sc as plsc

N = 100000
E = 1600000
G = 512
C14 = 14
NC, NS, L = 2, 16, 16

CH = 1024
KR = CH // 128
N_CHUNK = 98
EPT = CH * N_CHUNK
E_PAD = EPT * NS
EROWS = E_PAD // 128

DEG_KR = 8
DEG_CHUNKS = 49

ACC_DUMMY = 128
ACC_ROWS = N + ACC_DUMMY
N_PAD = 100096
RPT = N_PAD // NS

_mesh = plsc.VectorSubcoreMesh(
    core_axis_name="c", subcore_axis_name="s", num_cores=NC, num_subcores=NS)
_sc_params = pltpu.CompilerParams(use_tc_tiling_on_sc=False)

f32 = jnp.float32
i32 = jnp.int32


def _deg_kernel(dst2d, ones_hbm, zeros_hbm, deg0, deg1, acc, idxv, onev, sem):
    del sem
    c = lax.axis_index("c")
    t = lax.axis_index("s")
    pltpu.sync_copy(ones_hbm, onev)
    pltpu.sync_copy(zeros_hbm, acc.at[pl.ds(t * RPT, RPT)])
    plsc.subcore_barrier()
    row0 = c * (EROWS // 2) + t * (DEG_CHUNKS * DEG_KR)

    def chunk(i, carry):
        ro = row0 + i * DEG_KR
        pltpu.sync_copy(dst2d.at[pl.ds(ro, DEG_KR)], idxv)
        for j in range(DEG_KR):
            pltpu.sync_copy(onev, acc.at[idxv.at[j]], add=True)
        return carry

    lax.fori_loop(0, DEG_CHUNKS, chunk, 0)
    plsc.subcore_barrier()

    @pl.when(c == 0)
    def _():
        pltpu.sync_copy(acc.at[pl.ds(t * RPT, RPT)], deg0.at[pl.ds(t * RPT, RPT)])

    @pl.when(c == 1)
    def _():
        pltpu.sync_copy(acc.at[pl.ds(t * RPT, RPT)], deg1.at[pl.ds(t * RPT, RPT)])


def _sc_deg(dst2d, ones_c, zeros_c):
    return pl.kernel(
        _deg_kernel,
        out_type=[jax.ShapeDtypeStruct((N_PAD, L), f32)] * 2,
        mesh=_mesh,
        compiler_params=_sc_params,
        scratch_types=[
            pltpu.VMEM_SHARED((ACC_ROWS, L), f32),
            pltpu.VMEM((DEG_KR, 128), i32),
            pltpu.VMEM((128, L), f32),
            pltpu.SemaphoreType.DMA,
        ],
    )(dst2d, ones_c, zeros_c)


def _agg_kernel(nsl, src2d, dst2d, *refs):
    us = refs[:nsl]
    outs = refs[nsl:2 * nsl]
    acc, srcv, dstv, rows, sem = refs[2 * nsl:]
    c = lax.axis_index("c")
    t = lax.axis_index("s")
    spc = nsl // NC

    for s in range(nsl):
        @pl.when(c == s // spc)
        def _(s=s):
            pltpu.sync_copy(us[s].at[pl.ds(t * RPT, RPT)],
                            acc.at[pl.ds(t * RPT, RPT)])
            plsc.subcore_barrier()
            row0 = t * (N_CHUNK * KR)

            def chunk(i, carry):
                ro = row0 + i * KR
                pltpu.sync_copy(src2d.at[pl.ds(ro, KR)], srcv)
                pltpu.sync_copy(dst2d.at[pl.ds(ro, KR)], dstv)
                cps = [
                    pltpu.async_copy(us[s].at[srcv.at[j]],
                                     rows.at[pl.ds(j * 128, 128)], sem)
                    for j in range(KR)
                ]
                for cp in cps:
                    cp.wait()
                for j in range(KR):
                    pltpu.sync_copy(rows.at[pl.ds(j * 128, 128)],
                                    acc.at[dstv.at[j]], add=True)
                return carry

            lax.fori_loop(0, N_CHUNK, chunk, 0)
            plsc.subcore_barrier()
            pltpu.sync_copy(acc.at[pl.ds(t * RPT, RPT)],
                            outs[s].at[pl.ds(t * RPT, RPT)])


def _sc_agg(src2d, dst2d, u_slices):
    nsl = len(u_slices)
    return pl.kernel(
        functools.partial(_agg_kernel, nsl),
        out_type=[jax.ShapeDtypeStruct((N_PAD, L), f32)] * nsl,
        mesh=_mesh,
        compiler_params=_sc_params,
        scratch_types=[
            pltpu.VMEM_SHARED((ACC_ROWS, L), f32),
            pltpu.VMEM((KR, 128), i32),
            pltpu.VMEM((KR, 128), i32),
            pltpu.VMEM((CH, L), f32),
            pltpu.SemaphoreType.DMA,
        ],
    )(src2d, dst2d, *u_slices)


_R = 3128
_NB = N_PAD // _R


def _prep_body(x_ref, d0_ref, d1_ref, u0_ref, u1_ref, dinv_ref):
    deg = d0_ref[:, 0:1] + d1_ref[:, 0:1] + 1.0
    dinv = lax.rsqrt(deg)
    u = x_ref[...] * dinv
    u0_ref[...] = u[:, 0:16]
    u1_ref[...] = jnp.concatenate(
        [u[:, 16:20], jnp.zeros((_R, 12), f32)], axis=1)
    dinv_ref[...] = dinv


def _tc_prep(x, deg0, deg1):
    bs = lambda w: pl.BlockSpec((_R, w), lambda i: (i, 0))
    return pl.pallas_call(
        _prep_body,
        grid=(_NB,),
        in_specs=[bs(20), bs(L), bs(L)],
        out_specs=[bs(L), bs(L), bs(1)],
        out_shape=[
            jax.ShapeDtypeStruct((N_PAD, L), f32),
            jax.ShapeDtypeStruct((N_PAD, L), f32),
            jax.ShapeDtypeStruct((N_PAD, 1), f32),
        ],
    )(x, deg0, deg1)


def _mm_body(nin, nout, *refs):
    w_ref, b_ref, dinv_ref = refs[:3]
    zin = refs[3:3 + nin]
    uout = refs[3 + nin:]
    z = jnp.concatenate([r[...] for r in zin], axis=1)
    dinv = dinv_ref[...]
    t = jnp.dot(z * dinv, w_ref[...], preferred_element_type=f32) + b_ref[...]
    u = jnp.maximum(t, 0.0) * dinv
    for k, r in enumerate(uout):
        r[...] = u[:, L * k:L * (k + 1)]


def _tc_mm(z_slices, dinv, w, b):
    nin = len(z_slices)
    nout = w.shape[1] // L
    din = nin * L
    bs = lambda w_, : pl.BlockSpec((_R, w_), lambda i: (i, 0))
    full = lambda a, b_: pl.BlockSpec((a, b_), lambda i: (0, 0))
    return pl.pallas_call(
        functools.partial(_mm_body, nin, nout),
        grid=(_NB,),
        in_specs=[full(din, w.shape[1]), full(1, w.shape[1]), bs(1)]
        + [bs(L)] * nin,
        out_specs=[bs(L)] * nout,
        out_shape=[jax.ShapeDtypeStruct((N_PAD, L), f32)] * nout,
    )(w, b, dinv, *z_slices)


def _final_body(batch_ref, dinv_ref, w3_ref, b3_ref, wl_ref, bl_ref,
                z0, z1, z2, z3, out_ref, accp, accc):
    i = pl.program_id(0)

    @pl.when(i == 0)
    def _():
        accp[...] = jnp.zeros((G, 64), f32)
        accc[...] = jnp.zeros((G, 1), f32)

    z = jnp.concatenate([z0[...], z1[...], z2[...], z3[...]], axis=1)
    y = z * dinv_ref[...]
    b = batch_ref[...]
    iota = lax.broadcasted_iota(i32, (_R, G), 1)
    row = lax.broadcasted_iota(i32, (_R, 1), 0) + i * _R
    p = ((b == iota) & (row < N)).astype(f32)
    accp[...] += lax.dot_general(p, y, (((0,), (0,)), ((), ())),
                                 preferred_element_type=f32)
    accc[...] += lax.dot_general(p, jnp.ones((_R, 1), f32),
                                 (((0,), (0,)), ((), ())),
                                 preferred_element_type=f32)

    @pl.when(i == _NB - 1)
    def _():
        cnt = accc[...]
        pooled = accp[...] / jnp.maximum(cnt, 1.0)
        nonempty = (cnt > 0.0).astype(f32)
        h3 = jnp.dot(pooled, w3_ref[...], preferred_element_type=f32) \
            + b3_ref[...] * nonempty
        logits = jnp.dot(h3, wl_ref[...], preferred_element_type=f32) \
            + bl_ref[...]
        m = jnp.max(logits, axis=1, keepdims=True)
        lse = m + jnp.log(jnp.sum(jnp.exp(logits - m), axis=1, keepdims=True))
        out_ref[...] = logits - lse


def _tc_final(batch2d, dinv, w3, b3, wl, bl, z_slices):
    bs = lambda w_: pl.BlockSpec((_R, w_), lambda i: (i, 0))
    full = lambda a, b_: pl.BlockSpec((a, b_), lambda i: (0, 0))
    return pl.pallas_call(
        _final_body,
        grid=(_NB,),
        in_specs=[bs(1), bs(1), full(64, 128), full(1, 128), full(128, C14),
                  full(1, C14)] + [bs(L)] * 4,
        out_specs=full(G, C14),
        out_shape=jax.ShapeDtypeStruct((G, C14), f32),
        scratch_shapes=[pltpu.VMEM((G, 64), f32), pltpu.VMEM((G, 1), f32)],
    )(batch2d, dinv, w3, b3, wl, bl, *z_slices)


def kernel(x, edge_index, batch, W1, b1, W2, b2, W3, b3, Wl, bl):
    src = edge_index[0].astype(i32)
    dst = edge_index[1].astype(i32)
    pad = E_PAD - E
    src_p = jnp.concatenate([src, jnp.zeros((pad,), i32)])
    dst_p = jnp.concatenate(
        [dst, N + (jnp.arange(pad, dtype=i32) % ACC_DUMMY)])
    src2d = src_p.reshape(EROWS, 128)
    dst2d = dst_p.reshape(EROWS, 128)

    ones_c = jnp.ones((128, L), f32)
    zeros_c = jnp.zeros((RPT, L), f32)

    deg0, deg1 = _sc_deg(dst2d, ones_c, zeros_c)
    u10, u11, dinv = _tc_prep(x, deg0, deg1)

    z1 = _sc_agg(src2d, dst2d, [u10, u11])
    W1p = jnp.concatenate([W1, jnp.zeros((12, 64), f32)], axis=0)
    u2 = _tc_mm(z1, dinv, W1p, b1.reshape(1, 64))

    z2 = _sc_agg(src2d, dst2d, u2)
    u3 = _tc_mm(z2, dinv, W2, b2.reshape(1, 64))

    z3 = _sc_agg(src2d, dst2d, u3)
    return _tc_final(batch.reshape(N, 1), dinv, W3, b3.reshape(1, 128),
                     Wl, bl.reshape(1, C14), z3)


# speedup vs baseline: 15.2571x; 1.0087x over previous
import functools

import jax
import jax.numpy as jnp
from jax import lax
from jax.experimental import pallas as pl
from jax.experimental.pallas import tpu as pltpu
from jax.experimental.pallas import tpu_sc as plsc

N = 100000
E = 1600000
G = 512
C14 = 14
NC, NS, L = 2, 16, 16

CH = 768
N_CHUNK = 132
EPT = CH * N_CHUNK
E_PAD = EPT * NS

DEG_CH = 768
DEG_CHUNKS = 66

ACC_DUMMY = 128
ACC_ROWS = N + ACC_DUMMY
N_PAD = 100096
RPT = N_PAD // NS

_mesh = plsc.VectorSubcoreMesh(
    core_axis_name="c", subcore_axis_name="s", num_cores=NC, num_subcores=NS)
_sc_params = pltpu.CompilerParams(use_tc_tiling_on_sc=False)

f32 = jnp.float32
i32 = jnp.int32


def _deg_kernel(dst1d, ones_hbm, zeros_hbm, deg0, deg1, acc, idxv, onev, sem):
    del sem
    c = lax.axis_index("c")
    t = lax.axis_index("s")
    pltpu.sync_copy(ones_hbm, onev)
    pltpu.sync_copy(zeros_hbm, acc.at[pl.ds(t * RPT, RPT)])
    plsc.subcore_barrier()
    e0 = c * (E_PAD // 2) + t * (DEG_CHUNKS * DEG_CH)

    def chunk(i, carry):
        eo = e0 + i * DEG_CH
        pltpu.sync_copy(dst1d.at[pl.ds(eo, DEG_CH)], idxv)
        pltpu.sync_copy(onev, acc.at[idxv], add=True)
        return carry

    lax.fori_loop(0, DEG_CHUNKS, chunk, 0)
    plsc.subcore_barrier()

    @pl.when(c == 0)
    def _():
        pltpu.sync_copy(acc.at[pl.ds(t * RPT, RPT)], deg0.at[pl.ds(t * RPT, RPT)])

    @pl.when(c == 1)
    def _():
        pltpu.sync_copy(acc.at[pl.ds(t * RPT, RPT)], deg1.at[pl.ds(t * RPT, RPT)])


def _sc_deg(dst1d, ones_c, zeros_c):
    return pl.kernel(
        _deg_kernel,
        out_type=[jax.ShapeDtypeStruct((N_PAD, L), f32)] * 2,
        mesh=_mesh,
        compiler_params=_sc_params,
        scratch_types=[
            pltpu.VMEM_SHARED((ACC_ROWS, L), f32),
            pltpu.VMEM((DEG_CH,), i32),
            pltpu.VMEM((DEG_CH, L), f32),
            pltpu.SemaphoreType.DMA,
        ],
    )(dst1d, ones_c, zeros_c)


def _agg_kernel(nsl, src1d, dst1d, *refs):
    us = refs[:nsl]
    outs = refs[nsl:2 * nsl]
    acc, srcv, dstv, rows, srcv2, dstv2, rows2, sem, sem2 = refs[2 * nsl:]
    c = lax.axis_index("c")
    t = lax.axis_index("s")
    spc = nsl // NC

    for s in range(nsl):
        @pl.when(c == s // spc)
        def _(s=s):
            pltpu.sync_copy(us[s].at[pl.ds(t * RPT, RPT)],
                            acc.at[pl.ds(t * RPT, RPT)])
            plsc.subcore_barrier()
            e0 = t * EPT

            def chunk2(i, carry):
                eo = e0 + (2 * i) * CH
                pltpu.sync_copy(src1d.at[pl.ds(eo, CH)], srcv)
                pltpu.sync_copy(dst1d.at[pl.ds(eo, CH)], dstv)
                ga = pltpu.async_copy(us[s].at[srcv], rows, sem)
                pltpu.sync_copy(src1d.at[pl.ds(eo + CH, CH)], srcv2)
                pltpu.sync_copy(dst1d.at[pl.ds(eo + CH, CH)], dstv2)
                ga.wait()
                gb = pltpu.async_copy(us[s].at[srcv2], rows2, sem)
                sa = pltpu.async_copy(rows, acc.at[dstv], sem2, add=True)
                gb.wait()
                sb = pltpu.async_copy(rows2, acc.at[dstv2], sem2, add=True)
                sa.wait()
                sb.wait()
                return carry

            lax.fori_loop(0, N_CHUNK // 2, chunk2, 0)
            plsc.subcore_barrier()
            pltpu.sync_copy(acc.at[pl.ds(t * RPT, RPT)],
                            outs[s].at[pl.ds(t * RPT, RPT)])


def _sc_agg(src1d, dst1d, u_slices):
    nsl = len(u_slices)
    return pl.kernel(
        functools.partial(_agg_kernel, nsl),
        out_type=[jax.ShapeDtypeStruct((N_PAD, L), f32)] * nsl,
        mesh=_mesh,
        compiler_params=_sc_params,
        scratch_types=[
            pltpu.VMEM_SHARED((ACC_ROWS, L), f32),
            pltpu.VMEM((CH,), i32),
            pltpu.VMEM((CH,), i32),
            pltpu.VMEM((CH, L), f32),
            pltpu.VMEM((CH,), i32),
            pltpu.VMEM((CH,), i32),
            pltpu.VMEM((CH, L), f32),
            pltpu.SemaphoreType.DMA,
            pltpu.SemaphoreType.DMA,
        ],
    )(src1d, dst1d, *u_slices)


_R = 3128
_NB = N_PAD // _R


def _prep_body(x_ref, d0_ref, d1_ref, u0_ref, u1_ref, dinv_ref):
    deg = d0_ref[:, 0:1] + d1_ref[:, 0:1] + 1.0
    dinv = lax.rsqrt(deg)
    u = x_ref[...] * dinv
    u0_ref[...] = u[:, 0:16]
    u1_ref[...] = jnp.concatenate(
        [u[:, 16:20], jnp.zeros((_R, 12), f32)], axis=1)
    dinv_ref[...] = dinv


def _tc_prep(x, deg0, deg1):
    bs = lambda w: pl.BlockSpec((_R, w), lambda i: (i, 0))
    return pl.pallas_call(
        _prep_body,
        grid=(_NB,),
        in_specs=[bs(20), bs(L), bs(L)],
        out_specs=[bs(L), bs(L), bs(1)],
        out_shape=[
            jax.ShapeDtypeStruct((N_PAD, L), f32),
            jax.ShapeDtypeStruct((N_PAD, L), f32),
            jax.ShapeDtypeStruct((N_PAD, 1), f32),
        ],
    )(x, deg0, deg1)


def _mm_body(nin, nout, *refs):
    w_ref, b_ref, dinv_ref = refs[:3]
    zin = refs[3:3 + nin]
    uout = refs[3 + nin:]
    z = jnp.concatenate([r[...] for r in zin], axis=1)
    dinv = dinv_ref[...]
    t = jnp.dot(z * dinv, w_ref[...], preferred_element_type=f32) + b_ref[...]
    u = jnp.maximum(t, 0.0) * dinv
    for k, r in enumerate(uout):
        r[...] = u[:, L * k:L * (k + 1)]


def _tc_mm(z_slices, dinv, w, b):
    nin = len(z_slices)
    nout = w.shape[1] // L
    din = nin * L
    bs = lambda w_, : pl.BlockSpec((_R, w_), lambda i: (i, 0))
    full = lambda a, b_: pl.BlockSpec((a, b_), lambda i: (0, 0))
    return pl.pallas_call(
        functools.partial(_mm_body, nin, nout),
        grid=(_NB,),
        in_specs=[full(din, w.shape[1]), full(1, w.shape[1]), bs(1)]
        + [bs(L)] * nin,
        out_specs=[bs(L)] * nout,
        out_shape=[jax.ShapeDtypeStruct((N_PAD, L), f32)] * nout,
    )(w, b, dinv, *z_slices)


def _final_body(batch_ref, dinv_ref, w3_ref, b3_ref, wl_ref, bl_ref,
                z0, z1, z2, z3, out_ref, accp, accc):
    i = pl.program_id(0)

    @pl.when(i == 0)
    def _():
        accp[...] = jnp.zeros((G, 64), f32)
        accc[...] = jnp.zeros((G, 1), f32)

    z = jnp.concatenate([z0[...], z1[...], z2[...], z3[...]], axis=1)
    y = z * dinv_ref[...]
    b = batch_ref[...]
    iota = lax.broadcasted_iota(i32, (_R, G), 1)
    row = lax.broadcasted_iota(i32, (_R, 1), 0) + i * _R
    p = ((b == iota) & (row < N)).astype(f32)
    accp[...] += lax.dot_general(p, y, (((0,), (0,)), ((), ())),
                                 preferred_element_type=f32)
    accc[...] += lax.dot_general(p, jnp.ones((_R, 1), f32),
                                 (((0,), (0,)), ((), ())),
                                 preferred_element_type=f32)

    @pl.when(i == _NB - 1)
    def _():
        cnt = accc[...]
        pooled = accp[...] / jnp.maximum(cnt, 1.0)
        nonempty = (cnt > 0.0).astype(f32)
        h3 = jnp.dot(pooled, w3_ref[...], preferred_element_type=f32) \
            + b3_ref[...] * nonempty
        logits = jnp.dot(h3, wl_ref[...], preferred_element_type=f32) \
            + bl_ref[...]
        m = jnp.max(logits, axis=1, keepdims=True)
        lse = m + jnp.log(jnp.sum(jnp.exp(logits - m), axis=1, keepdims=True))
        out_ref[...] = logits - lse


def _tc_final(batch2d, dinv, w3, b3, wl, bl, z_slices):
    bs = lambda w_: pl.BlockSpec((_R, w_), lambda i: (i, 0))
    full = lambda a, b_: pl.BlockSpec((a, b_), lambda i: (0, 0))
    return pl.pallas_call(
        _final_body,
        grid=(_NB,),
        in_specs=[bs(1), bs(1), full(64, 128), full(1, 128), full(128, C14),
                  full(1, C14)] + [bs(L)] * 4,
        out_specs=full(G, C14),
        out_shape=jax.ShapeDtypeStruct((G, C14), f32),
        scratch_shapes=[pltpu.VMEM((G, 64), f32), pltpu.VMEM((G, 1), f32)],
    )(batch2d, dinv, w3, b3, wl, bl, *z_slices)


def kernel(x, edge_index, batch, W1, b1, W2, b2, W3, b3, Wl, bl):
    src = edge_index[0].astype(i32)
    dst = edge_index[1].astype(i32)
    pad = E_PAD - E
    src_p = jnp.concatenate([src, jnp.zeros((pad,), i32)])
    dst_p = jnp.concatenate(
        [dst, N + (jnp.arange(pad, dtype=i32) % ACC_DUMMY)])


    ones_c = jnp.ones((DEG_CH, L), f32)
    zeros_c = jnp.zeros((RPT, L), f32)

    deg0, deg1 = _sc_deg(dst_p, ones_c, zeros_c)
    u10, u11, dinv = _tc_prep(x, deg0, deg1)

    z1 = _sc_agg(src_p, dst_p, [u10, u11])
    W1p = jnp.concatenate([W1, jnp.zeros((12, 64), f32)], axis=0)
    u2 = _tc_mm(z1, dinv, W1p, b1.reshape(1, 64))

    z2 = _sc_agg(src_p, dst_p, u2)
    u3 = _tc_mm(z2, dinv, W2, b2.reshape(1, 64))

    z3 = _sc_agg(src_p, dst_p, u3)
    return _tc_final(batch.reshape(N, 1), dinv, W3, b3.reshape(1, 128),
                     Wl, bl.reshape(1, C14), z3)


# speedup vs baseline: 15.5683x; 1.0204x over previous
import functools

import jax
import jax.numpy as jnp
from jax import lax
from jax.experimental import pallas as pl
from jax.experimental.pallas import tpu as pltpu
from jax.experimental.pallas import tpu_sc as plsc

N = 100000
E = 1600000
G = 512
C14 = 14
NC, NS, L = 2, 16, 16

CH = 768
N_CHUNK = 132
EPT = CH * N_CHUNK
E_PAD = EPT * NS

DEG_CH = 768
DEG_CHUNKS = 66

ACC_DUMMY = 128
ACC_ROWS = N + ACC_DUMMY
N_PAD = 100096
RPT = N_PAD // NS

_mesh = plsc.VectorSubcoreMesh(
    core_axis_name="c", subcore_axis_name="s", num_cores=NC, num_subcores=NS)
_sc_params = pltpu.CompilerParams(use_tc_tiling_on_sc=False)

f32 = jnp.float32
i32 = jnp.int32


def _deg_kernel(dst1d, ones_hbm, zeros_hbm, deg_all, acc, idxv, onev, sem):
    del sem
    c = lax.axis_index("c")
    t = lax.axis_index("s")
    pltpu.sync_copy(ones_hbm, onev)
    pltpu.sync_copy(zeros_hbm, acc.at[pl.ds(t * RPT, RPT)])
    plsc.subcore_barrier()
    e0 = c * (E_PAD // 2) + t * (DEG_CHUNKS * DEG_CH)

    def chunk(i, carry):
        eo = e0 + i * DEG_CH
        pltpu.sync_copy(dst1d.at[pl.ds(eo, DEG_CH)], idxv)
        pltpu.sync_copy(onev, acc.at[idxv], add=True)
        return carry

    lax.fori_loop(0, DEG_CHUNKS, chunk, 0)
    plsc.subcore_barrier()
    pltpu.sync_copy(acc.at[pl.ds(t * RPT, RPT)],
                    deg_all.at[c].at[pl.ds(t * RPT, RPT)])


def _sc_deg(dst1d, ones_c, zeros_c):
    return pl.kernel(
        _deg_kernel,
        out_type=jax.ShapeDtypeStruct((NC, N_PAD, L), f32),
        mesh=_mesh,
        compiler_params=_sc_params,
        scratch_types=[
            pltpu.VMEM_SHARED((ACC_ROWS, L), f32),
            pltpu.VMEM((DEG_CH,), i32),
            pltpu.VMEM((DEG_CH, L), f32),
            pltpu.SemaphoreType.DMA,
        ],
    )(dst1d, ones_c, zeros_c)


def _agg_kernel(nsl, src1d, dst1d, u_all, z_all,
                acc, srcv, dstv, rows, srcv2, dstv2, rows2, sem, sem2):
    c = lax.axis_index("c")
    t = lax.axis_index("s")
    spc = nsl // NC

    def slice_body(si, carry):
        s = c * spc + si
        u_s = u_all.at[s]
        pltpu.sync_copy(u_s.at[pl.ds(t * RPT, RPT)],
                        acc.at[pl.ds(t * RPT, RPT)])
        plsc.subcore_barrier()
        e0 = t * EPT

        def chunk2(i, carry2):
            eo = e0 + (2 * i) * CH
            pltpu.sync_copy(src1d.at[pl.ds(eo, CH)], srcv)
            pltpu.sync_copy(dst1d.at[pl.ds(eo, CH)], dstv)
            ga = pltpu.async_copy(u_s.at[srcv], rows, sem)
            pltpu.sync_copy(src1d.at[pl.ds(eo + CH, CH)], srcv2)
            pltpu.sync_copy(dst1d.at[pl.ds(eo + CH, CH)], dstv2)
            ga.wait()
            gb = pltpu.async_copy(u_s.at[srcv2], rows2, sem)
            sa = pltpu.async_copy(rows, acc.at[dstv], sem2, add=True)
            gb.wait()
            sb = pltpu.async_copy(rows2, acc.at[dstv2], sem2, add=True)
            sa.wait()
            sb.wait()
            return carry2

        lax.fori_loop(0, N_CHUNK // 2, chunk2, 0)
        plsc.subcore_barrier()
        pltpu.sync_copy(acc.at[pl.ds(t * RPT, RPT)],
                        z_all.at[s].at[pl.ds(t * RPT, RPT)])
        return carry

    lax.fori_loop(0, spc, slice_body, 0)


def _sc_agg(src1d, dst1d, u_all):
    nsl = u_all.shape[0]
    return pl.kernel(
        functools.partial(_agg_kernel, nsl),
        out_type=jax.ShapeDtypeStruct((nsl, N_PAD, L), f32),
        mesh=_mesh,
        compiler_params=_sc_params,
        scratch_types=[
            pltpu.VMEM_SHARED((ACC_ROWS, L), f32),
            pltpu.VMEM((CH,), i32),
            pltpu.VMEM((CH,), i32),
            pltpu.VMEM((CH, L), f32),
            pltpu.VMEM((CH,), i32),
            pltpu.VMEM((CH,), i32),
            pltpu.VMEM((CH, L), f32),
            pltpu.SemaphoreType.DMA,
            pltpu.SemaphoreType.DMA,
        ],
    )(src1d, dst1d, u_all)


_R = 3128
_NB = N_PAD // _R


def _prep_body(x_ref, d_ref, u_ref, dinv_ref):
    deg = d_ref[0][:, 0:1] + d_ref[1][:, 0:1] + 1.0
    dinv = lax.rsqrt(deg)
    u = x_ref[...] * dinv
    u_ref[0] = u[:, 0:16]
    u_ref[1] = jnp.concatenate(
        [u[:, 16:20], jnp.zeros((_R, 12), f32)], axis=1)
    dinv_ref[...] = dinv


def _tc_prep(x, deg_all):
    bs = lambda w: pl.BlockSpec((_R, w), lambda i: (i, 0))
    return pl.pallas_call(
        _prep_body,
        grid=(_NB,),
        in_specs=[bs(20), pl.BlockSpec((NC, _R, L), lambda i: (0, i, 0))],
        out_specs=[pl.BlockSpec((2, _R, L), lambda i: (0, i, 0)), bs(1)],
        out_shape=[
            jax.ShapeDtypeStruct((2, N_PAD, L), f32),
            jax.ShapeDtypeStruct((N_PAD, 1), f32),
        ],
    )(x, deg_all)


def _mm_body(nin, nout, w_ref, b_ref, dinv_ref, z_ref, u_ref):
    z = jnp.concatenate([z_ref[k] for k in range(nin)], axis=1)
    dinv = dinv_ref[...]
    t = jnp.dot(z * dinv, w_ref[...], preferred_element_type=f32) + b_ref[...]
    u = jnp.maximum(t, 0.0) * dinv
    for k in range(nout):
        u_ref[k] = u[:, L * k:L * (k + 1)]


def _tc_mm(z_all, dinv, w, b):
    nin = z_all.shape[0]
    nout = w.shape[1] // L
    full = lambda a, b_: pl.BlockSpec((a, b_), lambda i: (0, 0))
    return pl.pallas_call(
        functools.partial(_mm_body, nin, nout),
        grid=(_NB,),
        in_specs=[full(nin * L, w.shape[1]), full(1, w.shape[1]),
                  pl.BlockSpec((_R, 1), lambda i: (i, 0)),
                  pl.BlockSpec((nin, _R, L), lambda i: (0, i, 0))],
        out_specs=pl.BlockSpec((nout, _R, L), lambda i: (0, i, 0)),
        out_shape=jax.ShapeDtypeStruct((nout, N_PAD, L), f32),
    )(w, b, dinv, z_all)


def _final_body(batch_ref, dinv_ref, w3_ref, b3_ref, wl_ref, bl_ref,
                z_ref, out_ref, accp, accc):
    i = pl.program_id(0)

    @pl.when(i == 0)
    def _():
        accp[...] = jnp.zeros((G, 64), f32)
        accc[...] = jnp.zeros((G, 1), f32)

    z = jnp.concatenate([z_ref[k] for k in range(4)], axis=1)
    y = z * dinv_ref[...]
    b = batch_ref[...]
    iota = lax.broadcasted_iota(i32, (_R, G), 1)
    row = lax.broadcasted_iota(i32, (_R, 1), 0) + i * _R
    p = ((b == iota) & (row < N)).astype(f32)
    accp[...] += lax.dot_general(p, y, (((0,), (0,)), ((), ())),
                                 preferred_element_type=f32)
    accc[...] += lax.dot_general(p, jnp.ones((_R, 1), f32),
                                 (((0,), (0,)), ((), ())),
                                 preferred_element_type=f32)

    @pl.when(i == _NB - 1)
    def _():
        cnt = accc[...]
        pooled = accp[...] / jnp.maximum(cnt, 1.0)
        nonempty = (cnt > 0.0).astype(f32)
        h3 = jnp.dot(pooled, w3_ref[...], preferred_element_type=f32) \
            + b3_ref[...] * nonempty
        logits = jnp.dot(h3, wl_ref[...], preferred_element_type=f32) \
            + bl_ref[...]
        m = jnp.max(logits, axis=1, keepdims=True)
        lse = m + jnp.log(jnp.sum(jnp.exp(logits - m), axis=1, keepdims=True))
        out_ref[...] = logits - lse


def _tc_final(batch2d, dinv, w3, b3, wl, bl, z_all):
    bs = lambda w_: pl.BlockSpec((_R, w_), lambda i: (i, 0))
    full = lambda a, b_: pl.BlockSpec((a, b_), lambda i: (0, 0))
    return pl.pallas_call(
        _final_body,
        grid=(_NB,),
        in_specs=[bs(1), bs(1), full(64, 128), full(1, 128), full(128, C14),
                  full(1, C14),
                  pl.BlockSpec((4, _R, L), lambda i: (0, i, 0))],
        out_specs=full(G, C14),
        out_shape=jax.ShapeDtypeStruct((G, C14), f32),
        scratch_shapes=[pltpu.VMEM((G, 64), f32), pltpu.VMEM((G, 1), f32)],
    )(batch2d, dinv, w3, b3, wl, bl, z_all)


def kernel(x, edge_index, batch, W1, b1, W2, b2, W3, b3, Wl, bl):
    src = edge_index[0].astype(i32)
    dst = edge_index[1].astype(i32)
    pad = E_PAD - E
    src_p = jnp.concatenate([src, jnp.zeros((pad,), i32)])
    dst_p = jnp.concatenate(
        [dst, N + (jnp.arange(pad, dtype=i32) % ACC_DUMMY)])

    ones_c = jnp.ones((DEG_CH, L), f32)
    zeros_c = jnp.zeros((RPT, L), f32)

    deg_all = _sc_deg(dst_p, ones_c, zeros_c)
    u1, dinv = _tc_prep(x, deg_all)

    z1 = _sc_agg(src_p, dst_p, u1)
    W1p = jnp.concatenate([W1, jnp.zeros((12, 64), f32)], axis=0)
    u2 = _tc_mm(z1, dinv, W1p, b1.reshape(1, 64))

    z2 = _sc_agg(src_p, dst_p, u2)
    u3 = _tc_mm(z2, dinv, W2, b2.reshape(1, 64))

    z3 = _sc_agg(src_p, dst_p, u3)
    return _tc_final(batch.reshape(N, 1), dinv, W3, b3.reshape(1, 128),
                     Wl, bl.reshape(1, C14), z3)
